# baseline (speedup 1.0000x reference)
import jax
import jax.numpy as jnp
from jax import lax
from jax.experimental import pallas as pl
from jax.experimental.pallas import tpu as pltpu
from jax.experimental.pallas import tpu_sc as plsc

N_RESP = 100000
D = 32
S = 200
B = 4096
NC = 2
NS = 16
NW = NC * NS
BB = B // NW
GROUP = 4
NPAIR = S // (2 * GROUP)
TILE = 8 * BB
D8STRIDE = NW * TILE
SROW = (D // 8) * D8STRIDE
OUT_WORDS = S * SROW


def _body(respT_hbm, table_hbm, pos_hbm, out_hbm,
          idx_v, rowsA, rowsB, outA, outB, pos_v, gA, gB, sA, sB):
    wid = lax.axis_index("s") * NC + lax.axis_index("c")
    pltpu.sync_copy(pos_hbm, pos_v)
    pltpu.sync_copy(respT_hbm.at[:, pl.ds(wid * BB, BB)], idx_v)

    lane = lax.iota(jnp.int32, 16)
    idxd0 = lax.shift_right_logical(lane, 3) * TILE + (lane & 7) * BB
    idxd1 = idxd0 + 2 * TILE

    def fire_g(s, rows, sem):
        pltpu.async_copy(table_hbm.at[idx_v.at[s]], rows, sem)

    def drain_g(rows, sem):
        pltpu.make_async_copy(table_hbm.at[idx_v.at[0]], rows, sem).wait()

    def fire_st(s0, outbuf, sem):
        for sl in range(GROUP):
            for d8 in range(4):
                pltpu.async_copy(
                    outbuf.at[pl.ds((sl * 4 + d8) * TILE, TILE)],
                    out_hbm.at[pl.ds((s0 + sl) * SROW + d8 * D8STRIDE
                                     + wid * TILE, TILE)],
                    sem,
                )

    def drain_st(outbuf, sem):
        pltpu.make_async_copy(outbuf, out_hbm.at[pl.ds(0, GROUP * 4 * TILE)],
                              sem).wait()

    def compute(s, s_local, rows, outbuf):
        p0 = pos_v[s, pl.ds(0, 16)]
        p1 = pos_v[s, pl.ds(16, 16)]

        def bi_body(k, inner):
            for u in range(4):
                bi = k * 4 + u
                off = s_local * 4 * TILE + bi
                v0 = rows[bi, pl.ds(0, 16)] + p0
                v1 = rows[bi, pl.ds(16, 16)] + p1
                plsc.store_scatter(outbuf, [idxd0 + off], v0)
                plsc.store_scatter(outbuf, [idxd1 + off], v1)
            return inner

        lax.fori_loop(0, BB // 4, bi_body, 0)

    def run_group(s0, outbuf):
        for sl in range(GROUP):
            s = s0 + sl
            if sl % 2 == 0:
                rows, sem, nrows, nsem = rowsA, gA, rowsB, gB
            else:
                rows, sem, nrows, nsem = rowsB, gB, rowsA, gA
            drain_g(rows, sem)

            def _fire(s=s, nrows=nrows, nsem=nsem):
                fire_g(s + 1, nrows, nsem)

            pl.when(s + 1 < S)(_fire)
            compute(s, sl, rows, outbuf)

    fire_g(0, rowsA, gA)

    def pair_body(gp, carry):
        s0 = 2 * GROUP * gp

        pl.when(gp > 0)(lambda: drain_st(outA, sA))
        run_group(s0, outA)
        fire_st(s0, outA, sA)

        pl.when(gp > 0)(lambda: drain_st(outB, sB))
        run_group(s0 + GROUP, outB)
        fire_st(s0 + GROUP, outB, sB)
        return carry

    lax.fori_loop(0, NPAIR, pair_body, 0)
    drain_st(outA, sA)
    drain_st(outB, sB)


_sc_kernel = pl.kernel(
    _body,
    out_type=jax.ShapeDtypeStruct((OUT_WORDS,), jnp.float32),
    mesh=plsc.VectorSubcoreMesh(
        core_axis_name="c", subcore_axis_name="s", num_cores=NC, num_subcores=NS
    ),
    scratch_types=[
        pltpu.VMEM((S, BB), jnp.int32),
        pltpu.VMEM((BB, D), jnp.float32),
        pltpu.VMEM((BB, D), jnp.float32),
        pltpu.VMEM((GROUP * 4 * TILE,), jnp.float32),
        pltpu.VMEM((GROUP * 4 * TILE,), jnp.float32),
        pltpu.VMEM((S, D), jnp.float32),
        pltpu.SemaphoreType.DMA,
        pltpu.SemaphoreType.DMA,
        pltpu.SemaphoreType.DMA,
        pltpu.SemaphoreType.DMA,
    ],
    compiler_params=pltpu.CompilerParams(
        use_tc_tiling_on_sc=False, needs_layout_passes=False
    ),
)


def kernel(responses, response_table, position_table):
    respT = responses.astype(jnp.int32).T
    raw = _sc_kernel(respT, response_table, position_table)
    raw5 = raw.reshape(S, D // 8, B // BB, 8, BB)
    return raw5.transpose(2, 4, 0, 1, 3).reshape(B, S, D)
